# baseline (speedup 1.0000x reference)
import functools

import jax
import jax.numpy as jnp
from jax import lax
from jax.experimental import pallas as pl
from jax.experimental.pallas import tpu as pltpu
from jax.experimental.pallas import tpu_sc as plsc

N = 10000
E = 320000
D = 128
G = 64
C = 10

NC = 2
NS = 16
NW = NC * NS
LW = 16

CH = 128
PER_W = 10240
E_PAD = PER_W * NW
N_PAD = 10240
ROWS_PER_SUB = N_PAD // NS

_vmesh = plsc.VectorSubcoreMesh(core_axis_name="c", subcore_axis_name="s")


NCH = PER_W // CH
NCHT = E_PAD // CH


@functools.partial(
    pl.kernel,
    out_type=jax.ShapeDtypeStruct((NC, N_PAD, D), jnp.float32),
    mesh=_vmesh,
    scratch_types=[
        pltpu.VMEM((2, CH), jnp.int32),
        pltpu.VMEM((2, CH), jnp.int32),
        pltpu.VMEM((2, CH), jnp.int32),
        pltpu.VMEM((2, CH), jnp.int32),
        pltpu.VMEM((CH, D), jnp.float32),
        pltpu.VMEM((CH, D), jnp.float32),
        pltpu.VMEM_SHARED((N_PAD, D), jnp.float32),
        pltpu.SemaphoreType.DMA,
        pltpu.SemaphoreType.DMA,
        pltpu.SemaphoreType.DMA,
        pltpu.SemaphoreType.DMA,
        pltpu.SemaphoreType.DMA,
        pltpu.SemaphoreType.DMA,
        pltpu.SemaphoreType.DMA,
        pltpu.SemaphoreType.DMA,
    ],
)
def _sc_agg(h_hbm, ids_hbm, zeros_hbm, out_hbm,
            idx0, idx1, idx2, idx3, rows0, rows1, acc,
            isem0, isem1, isem2, isem3, gsem0, gsem1, ssem0, ssem1):
    c = lax.axis_index("c")
    s = lax.axis_index("s")
    r0 = s * ROWS_PER_SUB
    pltpu.sync_copy(zeros_hbm.at[pl.ds(r0, ROWS_PER_SUB)], acc.at[pl.ds(r0, ROWS_PER_SUB)])
    plsc.subcore_barrier()
    b0 = (c * NS + s) * NCH

    idx = (idx0, idx1, idx2, idx3)
    rows = (rows0, rows1)
    isem = (isem0, isem1, isem2, isem3)
    gsem = (gsem0, gsem1)
    ssem = (ssem0, ssem1)

    pltpu.sync_copy(ids_hbm.at[b0], idx[0])
    pltpu.sync_copy(ids_hbm.at[b0 + 1], idx[1])
    pltpu.async_copy(h_hbm.at[idx[0].at[0]], rows[0], gsem[0])

    def step(k, p, m, wait_prev=True, wait_idx=True, do_gather=True, do_fetch=True):
        q = 1 - p
        m1 = (m + 1) % 4
        m2 = (m + 2) % 4
        pltpu.make_async_copy(h_hbm.at[idx[m].at[0]], rows[p], gsem[p]).wait()
        pltpu.async_copy(rows[p], acc.at[idx[m].at[1]], ssem[p], add=True)
        if wait_prev:
            pltpu.make_async_copy(rows[q], acc.at[idx[(m + 3) % 4].at[1]], ssem[q]).wait()
        if do_gather:
            if wait_idx:
                pltpu.make_async_copy(ids_hbm.at[k + 1], idx[m1], isem[m1]).wait()
            pltpu.async_copy(h_hbm.at[idx[m1].at[0]], rows[q], gsem[q])
        if do_fetch:
            pltpu.async_copy(ids_hbm.at[k + 2], idx[m2], isem[m2])

    step(b0 + 0, 0, 0, wait_prev=False, wait_idx=False)
    step(b0 + 1, 1, 1)
    step(b0 + 2, 0, 2)
    step(b0 + 3, 1, 3)

    @pl.loop(4, NCH - 4, step=4)
    def _(i):
        step(b0 + i, 0, 0)
        step(b0 + i + 1, 1, 1)
        step(b0 + i + 2, 0, 2)
        step(b0 + i + 3, 1, 3)

    step(b0 + NCH - 4, 0, 0)
    step(b0 + NCH - 3, 1, 1)
    step(b0 + NCH - 2, 0, 2, do_fetch=False)
    step(b0 + NCH - 1, 1, 3, do_gather=False, do_fetch=False)
    pltpu.make_async_copy(rows[1], acc.at[idx[3].at[1]], ssem[1]).wait()

    plsc.subcore_barrier()
    pltpu.sync_copy(acc.at[pl.ds(r0, ROWS_PER_SUB)], out_hbm.at[c, pl.ds(r0, ROWS_PER_SUB)])


@functools.partial(
    pl.kernel,
    out_type=jax.ShapeDtypeStruct((NC, N_PAD, D), jnp.float32),
    mesh=_vmesh,
    scratch_types=[
        pltpu.VMEM((2, CH), jnp.int32),
        pltpu.VMEM((2, CH), jnp.int32),
        pltpu.VMEM((CH, D), jnp.float32),
        pltpu.VMEM_SHARED((N_PAD, D), jnp.float32),
        pltpu.SemaphoreType.DMA,
        pltpu.SemaphoreType.DMA,
    ],
)
def _sc_cnt(ids_hbm, ones_hbm, zeros_hbm, out_hbm, idx0, idx1, ones_v, acc,
            isem0, isem1):
    c = lax.axis_index("c")
    s = lax.axis_index("s")
    r0 = s * ROWS_PER_SUB
    pltpu.sync_copy(zeros_hbm.at[pl.ds(r0, ROWS_PER_SUB)], acc.at[pl.ds(r0, ROWS_PER_SUB)])
    pltpu.sync_copy(ones_hbm, ones_v)
    plsc.subcore_barrier()
    b0 = (c * NS + s) * NCH

    idx = (idx0, idx1)
    isem = (isem0, isem1)
    pltpu.sync_copy(ids_hbm.at[b0], idx[0])
    pltpu.async_copy(ids_hbm.at[b0 + 1], idx[1], isem[1])

    def step(k, p, has_next, has_next2):
        q = 1 - p
        if has_next:
            pltpu.make_async_copy(ids_hbm.at[k + 1], idx[q], isem[q]).wait()
        pltpu.sync_copy(ones_v, acc.at[idx[p].at[1]], add=True)
        if has_next2:
            pltpu.async_copy(ids_hbm.at[k + 2], idx[p], isem[p])

    @pl.loop(0, NCH - 2, step=2)
    def _(i):
        step(b0 + i, 0, True, True)
        step(b0 + i + 1, 1, True, True)

    step(b0 + NCH - 2, 0, True, False)
    step(b0 + NCH - 1, 1, False, False)

    plsc.subcore_barrier()
    pltpu.sync_copy(acc.at[pl.ds(r0, ROWS_PER_SUB)], out_hbm.at[c, pl.ds(r0, ROWS_PER_SUB)])


def _tc_inv_body(cnt_ref, o_ref):
    cnt = cnt_ref[0, :, 0:1] + cnt_ref[1, :, 0:1]
    o_ref[...] = 1.0 / jnp.maximum(cnt, 1.0)


def _tc_inv(cnt2):
    return pl.pallas_call(
        _tc_inv_body,
        grid=(N // TN,),
        in_specs=[pl.BlockSpec((NC, TN, D), lambda i: (0, i, 0))],
        out_specs=pl.BlockSpec((TN, 1), lambda i: (i, 0)),
        out_shape=jax.ShapeDtypeStruct((N, 1), jnp.float32),
    )(cnt2)


_NFULL = N // CH
_TAIL = N - _NFULL * CH
_GPS = G // NS


@functools.partial(
    pl.kernel,
    out_type=jax.ShapeDtypeStruct((NC, G, D), jnp.float32),
    mesh=_vmesh,
    scratch_types=[
        pltpu.VMEM((CH,), jnp.int32),
        pltpu.VMEM((CH, D), jnp.float32),
        pltpu.VMEM((_TAIL,), jnp.int32),
        pltpu.VMEM((_TAIL, D), jnp.float32),
        pltpu.VMEM_SHARED((G, D), jnp.float32),
        pltpu.SemaphoreType.DMA,
    ],
)
def _sc_pool(h_hbm, batch_hbm, zeros_hbm, out_hbm, b_v, rows_v, bt_v, rowst_v, acc, sem):
    c = lax.axis_index("c")
    s = lax.axis_index("s")
    wid = c * NS + s
    g0 = s * _GPS
    pltpu.sync_copy(zeros_hbm.at[pl.ds(0, _GPS)], acc.at[pl.ds(g0, _GPS)])
    plsc.subcore_barrier()

    @pl.loop(wid, _NFULL, step=NW)
    def _(j):
        n0 = j * CH
        pltpu.sync_copy(batch_hbm.at[pl.ds(n0, CH)], b_v)
        pltpu.sync_copy(h_hbm.at[pl.ds(n0, CH)], rows_v)
        pltpu.sync_copy(rows_v, acc.at[b_v], add=True)

    @pl.when(wid == NW - 1)
    def _():
        n0 = _NFULL * CH
        pltpu.sync_copy(batch_hbm.at[pl.ds(n0, _TAIL)], bt_v)
        pltpu.sync_copy(h_hbm.at[pl.ds(n0, _TAIL)], rowst_v)
        pltpu.sync_copy(rowst_v, acc.at[bt_v], add=True)

    plsc.subcore_barrier()
    pltpu.sync_copy(acc.at[pl.ds(g0, _GPS)], out_hbm.at[c, pl.ds(g0, _GPS)])


TN = 2000


def _tc_right_body(h_ref, wr_ref, b_ref, o_ref):
    dn = (((1,), (1,)), ((), ()))
    y = lax.dot_general(h_ref[...], wr_ref[...], dn, preferred_element_type=jnp.float32)
    o_ref[...] = y + b_ref[...]


def _tc_right(h, Wr, b):
    return pl.pallas_call(
        _tc_right_body,
        grid=(N // TN,),
        in_specs=[
            pl.BlockSpec((TN, D), lambda i: (i, 0)),
            pl.BlockSpec((D, D), lambda i: (0, 0)),
            pl.BlockSpec((1, D), lambda i: (0, 0)),
        ],
        out_specs=pl.BlockSpec((TN, D), lambda i: (i, 0)),
        out_shape=jax.ShapeDtypeStruct((N, D), jnp.float32),
    )(h, Wr, b.reshape(1, D))


def _tc_combine_body(agg_ref, inv_ref, wl_ref, r_ref, o_ref, *, relu):
    agg = agg_ref[0] + agg_ref[1]
    inv = inv_ref[...]
    dn = (((1,), (1,)), ((), ()))
    y = lax.dot_general(agg * inv, wl_ref[...], dn, preferred_element_type=jnp.float32)
    y = y + r_ref[...]
    o_ref[...] = jnp.maximum(y, 0.0) if relu else y


def _tc_combine(agg2, inv, Wl, r, relu):
    return pl.pallas_call(
        functools.partial(_tc_combine_body, relu=relu),
        grid=(N // TN,),
        in_specs=[
            pl.BlockSpec((NC, TN, D), lambda i: (0, i, 0)),
            pl.BlockSpec((TN, 1), lambda i: (i, 0)),
            pl.BlockSpec((D, D), lambda i: (0, 0)),
            pl.BlockSpec((TN, D), lambda i: (i, 0)),
        ],
        out_specs=pl.BlockSpec((TN, D), lambda i: (i, 0)),
        out_shape=jax.ShapeDtypeStruct((N, D), jnp.float32),
    )(agg2, inv, Wl, r)


def _tc_final_body(p_ref, w_ref, b_ref, o_ref):
    p = p_ref[0] + p_ref[1]
    dn = (((1,), (1,)), ((), ()))
    y = lax.dot_general(p, w_ref[...], dn, preferred_element_type=jnp.float32)
    o_ref[...] = y + b_ref[...]


def _tc_final(pooled2, lin_W, lin_b):
    return pl.pallas_call(
        _tc_final_body,
        out_shape=jax.ShapeDtypeStruct((G, C), jnp.float32),
    )(pooled2, lin_W, lin_b.reshape(1, C))


def kernel(x, edge_index, batch, W1l, b1, W1r, W2l, b2, W2r, W3l, b3, W3r,
           W4l, b4, W4r, W5l, b5, W5r, W6l, b6, W6r, W7l, b7, W7r, lin_W, lin_b):
    src = edge_index[0]
    dst = edge_index[1]
    pad = E_PAD - E
    src_p = jnp.concatenate([src, jnp.zeros((pad,), jnp.int32)])
    trash = N + (jnp.arange(pad, dtype=jnp.int32) % (N_PAD - N))
    dst_p = jnp.concatenate([dst, trash])
    ids = (jnp.stack([src_p, dst_p], axis=0)
           .reshape(2, NCHT, CH).transpose(1, 0, 2))
    zeros = jnp.zeros((N_PAD, D), jnp.float32)
    ones = jnp.ones((CH, D), jnp.float32)

    cnt2 = _sc_cnt(ids, ones, zeros)
    inv = _tc_inv(cnt2)

    convs = [(W1l, b1, W1r), (W2l, b2, W2r), (W3l, b3, W3r), (W4l, b4, W4r),
             (W5l, b5, W5r), (W6l, b6, W6r), (W7l, b7, W7r)]
    h = x
    for i, (Wl, b, Wr) in enumerate(convs):
        r = _tc_right(h, Wr, b)
        agg2 = _sc_agg(h, ids, zeros)
        h = _tc_combine(agg2, inv, Wl, r, relu=(i < 6))

    pooled2 = _sc_pool(h, batch, zeros)
    return _tc_final(pooled2, lin_W, lin_b)


# speedup vs baseline: 1.0661x; 1.0661x over previous
import functools

import jax
import jax.numpy as jnp
from jax import lax
from jax.experimental import pallas as pl
from jax.experimental.pallas import tpu as pltpu
from jax.experimental.pallas import tpu_sc as plsc

N = 10000
E = 320000
D = 128
G = 64
C = 10

NC = 2
NS = 16
NW = NC * NS
LW = 16

CH = 80
PER_W = 10240
E_PAD = PER_W * NW
N_PAD = 10240
ROWS_PER_SUB = N_PAD // NS

_vmesh = plsc.VectorSubcoreMesh(core_axis_name="c", subcore_axis_name="s")


NCH = PER_W // CH
NCHT = E_PAD // CH


@functools.partial(
    pl.kernel,
    out_type=jax.ShapeDtypeStruct((NC, N_PAD, D), jnp.float32),
    mesh=_vmesh,
    scratch_types=[
        pltpu.VMEM((2, CH), jnp.int32),
        pltpu.VMEM((2, CH), jnp.int32),
        pltpu.VMEM((2, CH), jnp.int32),
        pltpu.VMEM((2, CH), jnp.int32),
        pltpu.VMEM((CH, D), jnp.float32),
        pltpu.VMEM((CH, D), jnp.float32),
        pltpu.VMEM((CH, D), jnp.float32),
        pltpu.VMEM((CH, D), jnp.float32),
        pltpu.VMEM_SHARED((N_PAD, D), jnp.float32),
        pltpu.SemaphoreType.DMA,
        pltpu.SemaphoreType.DMA,
        pltpu.SemaphoreType.DMA,
        pltpu.SemaphoreType.DMA,
        pltpu.SemaphoreType.DMA,
        pltpu.SemaphoreType.DMA,
        pltpu.SemaphoreType.DMA,
        pltpu.SemaphoreType.DMA,
        pltpu.SemaphoreType.DMA,
        pltpu.SemaphoreType.DMA,
        pltpu.SemaphoreType.DMA,
        pltpu.SemaphoreType.DMA,
    ],
)
def _sc_agg(h_hbm, ids_hbm, zeros_hbm, out_hbm,
            idx0, idx1, idx2, idx3, rows0, rows1, rows2, rows3, acc,
            isem0, isem1, isem2, isem3, gsem0, gsem1, gsem2, gsem3,
            ssem0, ssem1, ssem2, ssem3):
    c = lax.axis_index("c")
    s = lax.axis_index("s")
    r0 = s * ROWS_PER_SUB
    pltpu.sync_copy(zeros_hbm.at[pl.ds(r0, ROWS_PER_SUB)], acc.at[pl.ds(r0, ROWS_PER_SUB)])
    plsc.subcore_barrier()
    b0 = (c * NS + s) * NCH

    idx = (idx0, idx1, idx2, idx3)
    rows = (rows0, rows1, rows2, rows3)
    isem = (isem0, isem1, isem2, isem3)
    gsem = (gsem0, gsem1, gsem2, gsem3)
    ssem = (ssem0, ssem1, ssem2, ssem3)

    pltpu.sync_copy(ids_hbm.at[b0], idx[0])
    pltpu.sync_copy(ids_hbm.at[b0 + 1], idx[1])
    pltpu.async_copy(ids_hbm.at[b0 + 2], idx[2], isem[2])
    pltpu.async_copy(h_hbm.at[idx[0].at[0]], rows[0], gsem[0])
    pltpu.async_copy(h_hbm.at[idx[1].at[0]], rows[1], gsem[1])

    def step(k, b, wait_prev=True, fetch3=True, gather2=True):
        b1 = (b + 3) % 4
        b2 = (b + 2) % 4
        pltpu.make_async_copy(h_hbm.at[idx[b].at[0]], rows[b], gsem[b]).wait()
        pltpu.async_copy(rows[b], acc.at[idx[b].at[1]], ssem[b], add=True)
        if wait_prev:
            pltpu.make_async_copy(rows[b1], acc.at[idx[b1].at[1]], ssem[b1]).wait()
        if fetch3:
            pltpu.async_copy(ids_hbm.at[k + 3], idx[b1], isem[b1])
        if gather2:
            pltpu.make_async_copy(ids_hbm.at[k + 2], idx[b2], isem[b2]).wait()
            pltpu.async_copy(h_hbm.at[idx[b2].at[0]], rows[b2], gsem[b2])

    step(b0 + 0, 0, wait_prev=False)
    step(b0 + 1, 1)
    step(b0 + 2, 2)
    step(b0 + 3, 3)

    @pl.loop(4, NCH - 4, step=4)
    def _(i):
        step(b0 + i, 0)
        step(b0 + i + 1, 1)
        step(b0 + i + 2, 2)
        step(b0 + i + 3, 3)

    step(b0 + NCH - 4, 0)
    step(b0 + NCH - 3, 1, fetch3=False)
    step(b0 + NCH - 2, 2, fetch3=False, gather2=False)
    step(b0 + NCH - 1, 3, fetch3=False, gather2=False)
    pltpu.make_async_copy(rows[3], acc.at[idx[3].at[1]], ssem[3]).wait()

    plsc.subcore_barrier()
    pltpu.sync_copy(acc.at[pl.ds(r0, ROWS_PER_SUB)], out_hbm.at[c, pl.ds(r0, ROWS_PER_SUB)])


@functools.partial(
    pl.kernel,
    out_type=jax.ShapeDtypeStruct((NC, N_PAD, D), jnp.float32),
    mesh=_vmesh,
    scratch_types=[
        pltpu.VMEM((2, CH), jnp.int32),
        pltpu.VMEM((2, CH), jnp.int32),
        pltpu.VMEM((CH, D), jnp.float32),
        pltpu.VMEM_SHARED((N_PAD, D), jnp.float32),
        pltpu.SemaphoreType.DMA,
        pltpu.SemaphoreType.DMA,
    ],
)
def _sc_cnt(ids_hbm, ones_hbm, zeros_hbm, out_hbm, idx0, idx1, ones_v, acc,
            isem0, isem1):
    c = lax.axis_index("c")
    s = lax.axis_index("s")
    r0 = s * ROWS_PER_SUB
    pltpu.sync_copy(zeros_hbm.at[pl.ds(r0, ROWS_PER_SUB)], acc.at[pl.ds(r0, ROWS_PER_SUB)])
    pltpu.sync_copy(ones_hbm, ones_v)
    plsc.subcore_barrier()
    b0 = (c * NS + s) * NCH

    idx = (idx0, idx1)
    isem = (isem0, isem1)
    pltpu.sync_copy(ids_hbm.at[b0], idx[0])
    pltpu.async_copy(ids_hbm.at[b0 + 1], idx[1], isem[1])

    def step(k, p, has_next, has_next2):
        q = 1 - p
        if has_next:
            pltpu.make_async_copy(ids_hbm.at[k + 1], idx[q], isem[q]).wait()
        pltpu.sync_copy(ones_v, acc.at[idx[p].at[1]], add=True)
        if has_next2:
            pltpu.async_copy(ids_hbm.at[k + 2], idx[p], isem[p])

    @pl.loop(0, NCH - 2, step=2)
    def _(i):
        step(b0 + i, 0, True, True)
        step(b0 + i + 1, 1, True, True)

    step(b0 + NCH - 2, 0, True, False)
    step(b0 + NCH - 1, 1, False, False)

    plsc.subcore_barrier()
    pltpu.sync_copy(acc.at[pl.ds(r0, ROWS_PER_SUB)], out_hbm.at[c, pl.ds(r0, ROWS_PER_SUB)])


def _tc_inv_body(cnt_ref, o_ref):
    cnt = cnt_ref[0, :, 0:1] + cnt_ref[1, :, 0:1]
    o_ref[...] = 1.0 / jnp.maximum(cnt, 1.0)


def _tc_inv(cnt2):
    return pl.pallas_call(
        _tc_inv_body,
        grid=(N // TN,),
        in_specs=[pl.BlockSpec((NC, TN, D), lambda i: (0, i, 0))],
        out_specs=pl.BlockSpec((TN, 1), lambda i: (i, 0)),
        out_shape=jax.ShapeDtypeStruct((N, 1), jnp.float32),
    )(cnt2)


_NFULL = N // CH
_GPS = G // NS


@functools.partial(
    pl.kernel,
    out_type=jax.ShapeDtypeStruct((NC, G, D), jnp.float32),
    mesh=_vmesh,
    scratch_types=[
        pltpu.VMEM((CH,), jnp.int32),
        pltpu.VMEM((CH, D), jnp.float32),
        pltpu.VMEM_SHARED((G, D), jnp.float32),
        pltpu.SemaphoreType.DMA,
    ],
)
def _sc_pool(h_hbm, batch_hbm, zeros_hbm, out_hbm, b_v, rows_v, acc, sem):
    c = lax.axis_index("c")
    s = lax.axis_index("s")
    wid = c * NS + s
    g0 = s * _GPS
    pltpu.sync_copy(zeros_hbm.at[pl.ds(0, _GPS)], acc.at[pl.ds(g0, _GPS)])
    plsc.subcore_barrier()

    @pl.loop(wid, _NFULL, step=NW)
    def _(j):
        n0 = j * CH
        pltpu.sync_copy(batch_hbm.at[pl.ds(n0, CH)], b_v)
        pltpu.sync_copy(h_hbm.at[pl.ds(n0, CH)], rows_v)
        pltpu.sync_copy(rows_v, acc.at[b_v], add=True)

    plsc.subcore_barrier()
    pltpu.sync_copy(acc.at[pl.ds(g0, _GPS)], out_hbm.at[c, pl.ds(g0, _GPS)])


TN = 2000


def _tc_right_body(h_ref, wr_ref, b_ref, o_ref):
    dn = (((1,), (1,)), ((), ()))
    y = lax.dot_general(h_ref[...], wr_ref[...], dn, preferred_element_type=jnp.float32)
    o_ref[...] = y + b_ref[...]


def _tc_right(h, Wr, b):
    return pl.pallas_call(
        _tc_right_body,
        grid=(N // TN,),
        in_specs=[
            pl.BlockSpec((TN, D), lambda i: (i, 0)),
            pl.BlockSpec((D, D), lambda i: (0, 0)),
            pl.BlockSpec((1, D), lambda i: (0, 0)),
        ],
        out_specs=pl.BlockSpec((TN, D), lambda i: (i, 0)),
        out_shape=jax.ShapeDtypeStruct((N, D), jnp.float32),
    )(h, Wr, b.reshape(1, D))


def _tc_combine_body(agg_ref, inv_ref, wl_ref, r_ref, o_ref, *, relu):
    agg = agg_ref[0] + agg_ref[1]
    inv = inv_ref[...]
    dn = (((1,), (1,)), ((), ()))
    y = lax.dot_general(agg * inv, wl_ref[...], dn, preferred_element_type=jnp.float32)
    y = y + r_ref[...]
    o_ref[...] = jnp.maximum(y, 0.0) if relu else y


def _tc_combine(agg2, inv, Wl, r, relu):
    return pl.pallas_call(
        functools.partial(_tc_combine_body, relu=relu),
        grid=(N // TN,),
        in_specs=[
            pl.BlockSpec((NC, TN, D), lambda i: (0, i, 0)),
            pl.BlockSpec((TN, 1), lambda i: (i, 0)),
            pl.BlockSpec((D, D), lambda i: (0, 0)),
            pl.BlockSpec((TN, D), lambda i: (i, 0)),
        ],
        out_specs=pl.BlockSpec((TN, D), lambda i: (i, 0)),
        out_shape=jax.ShapeDtypeStruct((N, D), jnp.float32),
    )(agg2, inv, Wl, r)


def _tc_final_body(p_ref, w_ref, b_ref, o_ref):
    p = p_ref[0] + p_ref[1]
    dn = (((1,), (1,)), ((), ()))
    y = lax.dot_general(p, w_ref[...], dn, preferred_element_type=jnp.float32)
    o_ref[...] = y + b_ref[...]


def _tc_final(pooled2, lin_W, lin_b):
    return pl.pallas_call(
        _tc_final_body,
        out_shape=jax.ShapeDtypeStruct((G, C), jnp.float32),
    )(pooled2, lin_W, lin_b.reshape(1, C))


def kernel(x, edge_index, batch, W1l, b1, W1r, W2l, b2, W2r, W3l, b3, W3r,
           W4l, b4, W4r, W5l, b5, W5r, W6l, b6, W6r, W7l, b7, W7r, lin_W, lin_b):
    src = edge_index[0]
    dst = edge_index[1]
    pad = E_PAD - E
    src_p = jnp.concatenate([src, jnp.zeros((pad,), jnp.int32)])
    trash = N + (jnp.arange(pad, dtype=jnp.int32) % (N_PAD - N))
    dst_p = jnp.concatenate([dst, trash])
    ids = (jnp.stack([src_p, dst_p], axis=0)
           .reshape(2, NCHT, CH).transpose(1, 0, 2))
    zeros = jnp.zeros((N_PAD, D), jnp.float32)
    ones = jnp.ones((CH, D), jnp.float32)

    cnt2 = _sc_cnt(ids, ones, zeros)
    inv = _tc_inv(cnt2)

    convs = [(W1l, b1, W1r), (W2l, b2, W2r), (W3l, b3, W3r), (W4l, b4, W4r),
             (W5l, b5, W5r), (W6l, b6, W6r), (W7l, b7, W7r)]
    h = x
    for i, (Wl, b, Wr) in enumerate(convs):
        r = _tc_right(h, Wr, b)
        agg2 = _sc_agg(h, ids, zeros)
        h = _tc_combine(agg2, inv, Wl, r, relu=(i < 6))

    pooled2 = _sc_pool(h, batch, zeros)
    return _tc_final(pooled2, lin_W, lin_b)


# speedup vs baseline: 2.1635x; 2.0294x over previous
import functools

import jax
import jax.numpy as jnp
from jax import lax
from jax.experimental import pallas as pl
from jax.experimental.pallas import tpu as pltpu
from jax.experimental.pallas import tpu_sc as plsc

N = 10000
E = 320000
D = 128
G = 64
C = 10

NC = 2
NS = 16
NW = NC * NS

CH = 64
E_PAD = 327680
NT = 3456
NPASS = 3
N_PAD = NPASS * NT
ROWS_PER_SUB = N_PAD // NS
TROWS_PER_SUB = NT // NS

E_CAP = 114688
NCHP = E_CAP // CH // NW
NCHT = E_PAD // CH
NCH_CNT = NCHT // NW

CHP = 80

_vmesh = plsc.VectorSubcoreMesh(core_axis_name="c", subcore_axis_name="s")


@functools.partial(
    pl.kernel,
    out_type=jax.ShapeDtypeStruct((NC, N_PAD, D), jnp.float32),
    mesh=_vmesh,
    scratch_types=[
        pltpu.VMEM((2, CH), jnp.int32),
        pltpu.VMEM((2, CH), jnp.int32),
        pltpu.VMEM((2, CH), jnp.int32),
        pltpu.VMEM((2, CH), jnp.int32),
        pltpu.VMEM((CH, D), jnp.float32),
        pltpu.VMEM((CH, D), jnp.float32),
        pltpu.VMEM_SHARED((NT, D), jnp.float32),
        pltpu.VMEM_SHARED((N_PAD, D), jnp.float32),
        pltpu.SemaphoreType.DMA,
        pltpu.SemaphoreType.DMA,
        pltpu.SemaphoreType.DMA,
        pltpu.SemaphoreType.DMA,
        pltpu.SemaphoreType.DMA,
        pltpu.SemaphoreType.DMA,
        pltpu.SemaphoreType.DMA,
        pltpu.SemaphoreType.DMA,
    ],
)
def _sc_agg(h_hbm, ids0_hbm, ids1_hbm, ids2_hbm, zeros_hbm, out_hbm,
            idx0, idx1, idx2, idx3, rows0, rows1, h_sp, acc,
            isem0, isem1, isem2, isem3, gsem0, gsem1, ssem0, ssem1):
    c = lax.axis_index("c")
    s = lax.axis_index("s")
    r0 = s * ROWS_PER_SUB
    t0 = s * TROWS_PER_SUB
    pltpu.sync_copy(zeros_hbm.at[pl.ds(r0, ROWS_PER_SUB)], acc.at[pl.ds(r0, ROWS_PER_SUB)])
    b0 = (c * NS + s) * NCHP

    idx = (idx0, idx1, idx2, idx3)
    rows = (rows0, rows1)
    isem = (isem0, isem1, isem2, isem3)
    gsem = (gsem0, gsem1)
    ssem = (ssem0, ssem1)

    def one_pass(ids_hbm):
        pltpu.sync_copy(ids_hbm.at[b0], idx[0])
        pltpu.sync_copy(ids_hbm.at[b0 + 1], idx[1])
        pltpu.async_copy(h_sp.at[idx[0].at[0]], rows[0], gsem[0])

        def step(k, p, m, wait_prev=True, wait_idx=True, do_gather=True, do_fetch=True):
            q = 1 - p
            m1 = (m + 1) % 4
            m2 = (m + 2) % 4
            pltpu.make_async_copy(h_sp.at[idx[m].at[0]], rows[p], gsem[p]).wait()
            pltpu.async_copy(rows[p], acc.at[idx[m].at[1]], ssem[p], add=True)
            if wait_prev:
                pltpu.make_async_copy(rows[q], acc.at[idx[(m + 3) % 4].at[1]], ssem[q]).wait()
            if do_gather:
                if wait_idx:
                    pltpu.make_async_copy(ids_hbm.at[k + 1], idx[m1], isem[m1]).wait()
                pltpu.async_copy(h_sp.at[idx[m1].at[0]], rows[q], gsem[q])
            if do_fetch:
                pltpu.async_copy(ids_hbm.at[k + 2], idx[m2], isem[m2])

        step(b0 + 0, 0, 0, wait_prev=False, wait_idx=False)
        step(b0 + 1, 1, 1)
        step(b0 + 2, 0, 2)
        step(b0 + 3, 1, 3)

        @pl.loop(4, NCHP - 4, step=4)
        def _(i):
            step(b0 + i, 0, 0)
            step(b0 + i + 1, 1, 1)
            step(b0 + i + 2, 0, 2)
            step(b0 + i + 3, 1, 3)

        step(b0 + NCHP - 4, 0, 0)
        step(b0 + NCHP - 3, 1, 1)
        step(b0 + NCHP - 2, 0, 2, do_fetch=False)
        step(b0 + NCHP - 1, 1, 3, do_gather=False, do_fetch=False)
        pltpu.make_async_copy(rows[1], acc.at[idx[3].at[1]], ssem[1]).wait()

    for t, ids_hbm in enumerate((ids0_hbm, ids1_hbm, ids2_hbm)):
        pltpu.sync_copy(h_hbm.at[pl.ds(t * NT + t0, TROWS_PER_SUB)],
                        h_sp.at[pl.ds(t0, TROWS_PER_SUB)])
        plsc.subcore_barrier()
        one_pass(ids_hbm)
        plsc.subcore_barrier()

    pltpu.sync_copy(acc.at[pl.ds(r0, ROWS_PER_SUB)], out_hbm.at[c, pl.ds(r0, ROWS_PER_SUB)])


@functools.partial(
    pl.kernel,
    out_type=jax.ShapeDtypeStruct((NC, N_PAD, D), jnp.float32),
    mesh=_vmesh,
    scratch_types=[
        pltpu.VMEM((2, CH), jnp.int32),
        pltpu.VMEM((2, CH), jnp.int32),
        pltpu.VMEM((CH, D), jnp.float32),
        pltpu.VMEM_SHARED((N_PAD, D), jnp.float32),
        pltpu.SemaphoreType.DMA,
        pltpu.SemaphoreType.DMA,
    ],
)
def _sc_cnt(ids_hbm, ones_hbm, zeros_hbm, out_hbm, idx0, idx1, ones_v, acc,
            isem0, isem1):
    c = lax.axis_index("c")
    s = lax.axis_index("s")
    r0 = s * ROWS_PER_SUB
    pltpu.sync_copy(zeros_hbm.at[pl.ds(r0, ROWS_PER_SUB)], acc.at[pl.ds(r0, ROWS_PER_SUB)])
    pltpu.sync_copy(ones_hbm, ones_v)
    plsc.subcore_barrier()
    b0 = (c * NS + s) * NCH_CNT

    idx = (idx0, idx1)
    isem = (isem0, isem1)
    pltpu.sync_copy(ids_hbm.at[b0], idx[0])
    pltpu.async_copy(ids_hbm.at[b0 + 1], idx[1], isem[1])

    def step(k, p, has_next, has_next2):
        q = 1 - p
        if has_next:
            pltpu.make_async_copy(ids_hbm.at[k + 1], idx[q], isem[q]).wait()
        pltpu.sync_copy(ones_v, acc.at[idx[p].at[1]], add=True)
        if has_next2:
            pltpu.async_copy(ids_hbm.at[k + 2], idx[p], isem[p])

    @pl.loop(0, NCH_CNT - 2, step=2)
    def _(i):
        step(b0 + i, 0, True, True)
        step(b0 + i + 1, 1, True, True)

    step(b0 + NCH_CNT - 2, 0, True, False)
    step(b0 + NCH_CNT - 1, 1, False, False)

    plsc.subcore_barrier()
    pltpu.sync_copy(acc.at[pl.ds(r0, ROWS_PER_SUB)], out_hbm.at[c, pl.ds(r0, ROWS_PER_SUB)])


_NFULL = N // CHP
_GPS = G // NS


@functools.partial(
    pl.kernel,
    out_type=jax.ShapeDtypeStruct((NC, G, D), jnp.float32),
    mesh=_vmesh,
    scratch_types=[
        pltpu.VMEM((CHP,), jnp.int32),
        pltpu.VMEM((CHP, D), jnp.float32),
        pltpu.VMEM_SHARED((G, D), jnp.float32),
        pltpu.SemaphoreType.DMA,
    ],
)
def _sc_pool(h_hbm, batch_hbm, zeros_hbm, out_hbm, b_v, rows_v, acc, sem):
    c = lax.axis_index("c")
    s = lax.axis_index("s")
    wid = c * NS + s
    g0 = s * _GPS
    pltpu.sync_copy(zeros_hbm.at[pl.ds(0, _GPS)], acc.at[pl.ds(g0, _GPS)])
    plsc.subcore_barrier()

    @pl.loop(wid, _NFULL, step=NW)
    def _(j):
        n0 = j * CHP
        pltpu.sync_copy(batch_hbm.at[pl.ds(n0, CHP)], b_v)
        pltpu.sync_copy(h_hbm.at[pl.ds(n0, CHP)], rows_v)
        pltpu.sync_copy(rows_v, acc.at[b_v], add=True)

    plsc.subcore_barrier()
    pltpu.sync_copy(acc.at[pl.ds(g0, _GPS)], out_hbm.at[c, pl.ds(g0, _GPS)])


TN = 1296


def _tc_inv_body(cnt_ref, o_ref):
    cnt = cnt_ref[0, :, 0:1] + cnt_ref[1, :, 0:1]
    o_ref[...] = 1.0 / jnp.maximum(cnt, 1.0)


def _tc_inv(cnt2):
    return pl.pallas_call(
        _tc_inv_body,
        grid=(N_PAD // TN,),
        in_specs=[pl.BlockSpec((NC, TN, D), lambda i: (0, i, 0))],
        out_specs=pl.BlockSpec((TN, 1), lambda i: (i, 0)),
        out_shape=jax.ShapeDtypeStruct((N_PAD, 1), jnp.float32),
    )(cnt2)


def _tc_right_body(h_ref, wr_ref, b_ref, o_ref):
    dn = (((1,), (1,)), ((), ()))
    y = lax.dot_general(h_ref[...], wr_ref[...], dn, preferred_element_type=jnp.float32)
    o_ref[...] = y + b_ref[...]


def _tc_right(h, Wr, b):
    return pl.pallas_call(
        _tc_right_body,
        grid=(N_PAD // TN,),
        in_specs=[
            pl.BlockSpec((TN, D), lambda i: (i, 0)),
            pl.BlockSpec((D, D), lambda i: (0, 0)),
            pl.BlockSpec((1, D), lambda i: (0, 0)),
        ],
        out_specs=pl.BlockSpec((TN, D), lambda i: (i, 0)),
        out_shape=jax.ShapeDtypeStruct((N_PAD, D), jnp.float32),
    )(h, Wr, b.reshape(1, D))


def _tc_combine_body(agg_ref, inv_ref, wl_ref, r_ref, o_ref, *, relu):
    agg = agg_ref[0] + agg_ref[1]
    inv = inv_ref[...]
    dn = (((1,), (1,)), ((), ()))
    y = lax.dot_general(agg * inv, wl_ref[...], dn, preferred_element_type=jnp.float32)
    y = y + r_ref[...]
    o_ref[...] = jnp.maximum(y, 0.0) if relu else y


def _tc_combine(agg2, inv, Wl, r, relu):
    return pl.pallas_call(
        functools.partial(_tc_combine_body, relu=relu),
        grid=(N_PAD // TN,),
        in_specs=[
            pl.BlockSpec((NC, TN, D), lambda i: (0, i, 0)),
            pl.BlockSpec((TN, 1), lambda i: (i, 0)),
            pl.BlockSpec((D, D), lambda i: (0, 0)),
            pl.BlockSpec((TN, D), lambda i: (i, 0)),
        ],
        out_specs=pl.BlockSpec((TN, D), lambda i: (i, 0)),
        out_shape=jax.ShapeDtypeStruct((N_PAD, D), jnp.float32),
    )(agg2, inv, Wl, r)


def _tc_final_body(p_ref, w_ref, b_ref, o_ref):
    p = p_ref[0] + p_ref[1]
    dn = (((1,), (1,)), ((), ()))
    y = lax.dot_general(p, w_ref[...], dn, preferred_element_type=jnp.float32)
    o_ref[...] = y + b_ref[...]


def _tc_final(pooled2, lin_W, lin_b):
    return pl.pallas_call(
        _tc_final_body,
        out_shape=jax.ShapeDtypeStruct((G, C), jnp.float32),
    )(pooled2, lin_W, lin_b.reshape(1, C))


def kernel(x, edge_index, batch, W1l, b1, W1r, W2l, b2, W2r, W3l, b3, W3r,
           W4l, b4, W4r, W5l, b5, W5r, W6l, b6, W6r, W7l, b7, W7r, lin_W, lin_b):
    src = edge_index[0]
    dst = edge_index[1]
    pad = E_PAD - E
    src_p = jnp.concatenate([src, jnp.zeros((pad,), jnp.int32)])
    trash = N + (jnp.arange(E_PAD, dtype=jnp.int32) % (N_PAD - N))
    dst_p = jnp.concatenate([dst, trash[:pad]])
    ids = (jnp.stack([src_p, dst_p], axis=0)
           .reshape(2, NCHT, CH).transpose(1, 0, 2))

    eidx = jnp.arange(E_PAD)
    trash_cap = N + (jnp.arange(E_CAP, dtype=jnp.int32) % (N_PAD - N))
    ids_t = []
    for t in range(NPASS):
        in_t = (src_p >= t * NT) & (src_p < (t + 1) * NT) & (eidx < E)
        order = jnp.flatnonzero(in_t, size=E_CAP, fill_value=0)
        n_t = jnp.sum(in_t)
        valid = jnp.arange(E_CAP) < n_t
        src_t = jnp.where(valid, jnp.take(src_p, order) - t * NT, 0)
        dst_t = jnp.where(valid, jnp.take(dst_p, order), trash_cap)
        ids_t.append(jnp.stack([src_t, dst_t], axis=0)
                     .reshape(2, E_CAP // CH, CH).transpose(1, 0, 2))

    zeros = jnp.zeros((N_PAD, D), jnp.float32)
    ones = jnp.ones((CH, D), jnp.float32)

    cnt2 = _sc_cnt(ids, ones, zeros)
    inv = _tc_inv(cnt2)

    x_pad = jnp.concatenate([x, jnp.zeros((N_PAD - N, D), jnp.float32)])

    convs = [(W1l, b1, W1r), (W2l, b2, W2r), (W3l, b3, W3r), (W4l, b4, W4r),
             (W5l, b5, W5r), (W6l, b6, W6r), (W7l, b7, W7r)]
    h = x_pad
    for i, (Wl, b, Wr) in enumerate(convs):
        r = _tc_right(h, Wr, b)
        agg2 = _sc_agg(h, ids_t[0], ids_t[1], ids_t[2], zeros)
        h = _tc_combine(agg2, inv, Wl, r, relu=(i < 6))

    pooled2 = _sc_pool(h, batch, zeros)
    return _tc_final(pooled2, lin_W, lin_b)
